# baseline (speedup 1.0000x reference)
import functools

import jax
import jax.numpy as jnp
from jax import lax
from jax.experimental import pallas as pl
from jax.experimental.pallas import tpu as pltpu
from jax.experimental.pallas import tpu_sc as plsc

N_NODES = 50000
D = 512
E = 512
B = 10000
S = 10

NC, NS, L = 2, 16, 16
NW = NC * NS
B_PAD = 10240
K = 4
KS = K * S
NB = 4
PF = 3
NHALF = 2
B_H = B_PAD // NHALF
RPW = B_H // NW
NCHUNK = RPW // K


def _sc_gather_body(nodes_hbm, neigh_hbm, feat_hbm,
                    self_out, mean_out, *scr):
    wid = lax.axis_index("s") * NC + lax.axis_index("c")
    base = wid * RPW
    sidx, nidx = scr[0], scr[1]
    sbufs = scr[2:2 + NB]
    nbufs = scr[2 + NB:2 + 2 * NB]
    accs = scr[2 + 2 * NB:2 + 3 * NB]
    sem_gs = scr[2 + 3 * NB:2 + 4 * NB]
    sem_gn = scr[2 + 4 * NB:2 + 5 * NB]
    sem_w = scr[2 + 5 * NB:2 + 6 * NB]
    sem_ws = scr[2 + 6 * NB:2 + 7 * NB]

    pltpu.sync_copy(nodes_hbm.at[wid], sidx)
    pltpu.sync_copy(neigh_hbm.at[wid], nidx)

    inv_s = jnp.float32(1.0 / S)

    def issue_gathers(ch, q):
        pltpu.async_copy(feat_hbm.at[sidx.at[ch]], sbufs[q], sem_gs[q])
        pltpu.async_copy(feat_hbm.at[nidx.at[ch]], nbufs[q], sem_gn[q])

    for ch in range(PF):
        issue_gathers(ch, ch)

    def group_body(g, _):
        for par in range(NB):
            ch = g * NB + par
            row0 = base + ch * K
            sbuf, nbuf, acc = sbufs[par], nbufs[par], accs[par]
            pltpu.make_async_copy(feat_hbm.at[sidx.at[ch]], sbuf,
                                  sem_gs[par]).wait()
            pltpu.make_async_copy(feat_hbm.at[nidx.at[ch]], nbuf,
                                  sem_gn[par]).wait()
            pltpu.async_copy(sbuf, self_out.at[pl.ds(row0, K)], sem_ws[par])
            @pl.when(ch >= NB)
            def _():
                pltpu.make_async_copy(
                    acc, mean_out.at[pl.ds(row0, K)], sem_w[par]).wait()

            def col_body(c, _):
                sl = pl.ds(c * L, L)
                for r in range(K):
                    a = nbuf[r * S, sl]
                    for j in range(1, S):
                        a = a + nbuf[r * S + j, sl]
                    acc[r, sl] = a * inv_s
                return 0
            lax.fori_loop(0, D // L, col_body, 0)

            pltpu.async_copy(acc, mean_out.at[pl.ds(row0, K)], sem_w[par])

            q = (par + PF) % NB
            @pl.when(ch + PF < NCHUNK)
            def _():
                @pl.when(ch + PF >= NB)
                def _():
                    pltpu.make_async_copy(
                        sbufs[q], self_out.at[pl.ds(row0, K)],
                        sem_ws[q]).wait()
                issue_gathers(ch + PF, q)
        return 0

    lax.fori_loop(0, NCHUNK // NB, group_body, 0)

    for par in range(NB):
        pltpu.make_async_copy(accs[par], mean_out.at[pl.ds(0, K)],
                              sem_w[par]).wait()
        pltpu.make_async_copy(sbufs[par], self_out.at[pl.ds(0, K)],
                              sem_ws[par]).wait()


def _sc_gather(nodes_w, neigh_w, features):
    mesh = plsc.VectorSubcoreMesh(core_axis_name="c", subcore_axis_name="s")
    f = pl.kernel(
        _sc_gather_body,
        out_type=(
            jax.ShapeDtypeStruct((B_H, D), jnp.float32),
            jax.ShapeDtypeStruct((B_H, D), jnp.float32),
        ),
        mesh=mesh,
        scratch_types=[
            pltpu.VMEM((NCHUNK, K), jnp.int32),
            pltpu.VMEM((NCHUNK, KS), jnp.int32),
        ] + [pltpu.VMEM((K, D), jnp.float32)] * NB
          + [pltpu.VMEM((KS, D), jnp.float32)] * NB
          + [pltpu.VMEM((K, D), jnp.float32)] * NB
          + [pltpu.SemaphoreType.DMA] * (4 * NB),
    )
    return f(nodes_w, neigh_w, features)


def _mm_body(ws_ref, wn_ref, self_ref, mean_ref, out_ref):
    a = lax.dot_general(ws_ref[...], self_ref[...],
                        (((1,), (1,)), ((), ())),
                        preferred_element_type=jnp.float32)
    b = lax.dot_general(wn_ref[...], mean_ref[...],
                        (((1,), (1,)), ((), ())),
                        preferred_element_type=jnp.float32)
    out_ref[...] = jnp.maximum(a + b, 0.0)


BT = 512


def _tc_matmul(ws, wn, self_f, mean_f):
    grid = (B_H // BT,)
    return pl.pallas_call(
        _mm_body,
        grid=grid,
        in_specs=[
            pl.BlockSpec((E, D), lambda i: (0, 0)),
            pl.BlockSpec((E, D), lambda i: (0, 0)),
            pl.BlockSpec((BT, D), lambda i: (i, 0)),
            pl.BlockSpec((BT, D), lambda i: (i, 0)),
        ],
        out_specs=pl.BlockSpec((E, BT), lambda i: (0, i)),
        out_shape=jax.ShapeDtypeStruct((E, B_H), jnp.float32),
        compiler_params=pltpu.CompilerParams(
            dimension_semantics=("arbitrary",)),
    )(ws, wn, self_f, mean_f)


def kernel(nodes, neigh_idx, features, weight):
    nodes = nodes.astype(jnp.int32)
    neigh_idx = neigh_idx.astype(jnp.int32)
    pad_n = B_PAD - B
    pad_rows = (jnp.arange(pad_n, dtype=jnp.int32) * 37) % N_NODES
    nodes_p = jnp.concatenate([nodes, pad_rows])
    pad_rows2 = (jnp.arange(pad_n * S, dtype=jnp.int32) * 37) % N_NODES
    neigh_p = jnp.concatenate([neigh_idx.reshape(-1), pad_rows2])
    ws = weight[:, :D]
    wn = weight[:, D:]

    outs = []
    feats = []
    for h in range(NHALF):
        nodes_w = lax.dynamic_slice_in_dim(nodes_p, h * B_H, B_H).reshape(
            NW, NCHUNK, K)
        neigh_w = lax.dynamic_slice_in_dim(neigh_p, h * B_H * S,
                                           B_H * S).reshape(NW, NCHUNK, KS)
        feats.append(_sc_gather(nodes_w, neigh_w, features))
    for h in range(NHALF):
        self_f, mean_f = feats[h]
        outs.append(_tc_matmul(ws, wn, self_f, mean_f))
    return jnp.concatenate(outs, axis=1)[:, :B]
